# baseline (speedup 1.0000x reference)
import jax
import jax.numpy as jnp
from jax.experimental import pallas as pl
from jax.experimental.pallas import tpu as pltpu

_N_CLASSES = 100000
_SMOOTHING = 0.1
_CONFIDENCE = 1.0 - _SMOOTHING
_EPS = _SMOOTHING / (_N_CLASSES - 1)

_ROWS = 1024
_BLOCK_W = 2048
_NUM_BLOCKS = (_N_CLASSES + _BLOCK_W - 1) // _BLOCK_W


def _loss_kernel(lsm_ref, tgt_ref, out_ref):
    j = pl.program_id(0)
    blk = lsm_ref[...]
    col = jax.lax.broadcasted_iota(jnp.int32, (_ROWS, _BLOCK_W), 1) + j * _BLOCK_W
    blk = jnp.where(col < _N_CLASSES, blk, 0.0)
    s = jnp.sum(blk)
    tgt = tgt_ref[...]
    g = jnp.sum(jnp.where(col == tgt, blk, 0.0))
    out_ref[...] = jnp.reshape(_EPS * s + (_CONFIDENCE - _EPS) * g, (1, 1, 1))


def kernel(lsm, target):
    tgt = target.astype(jnp.int32).reshape(_ROWS, 1)
    partials = pl.pallas_call(
        _loss_kernel,
        grid=(_NUM_BLOCKS,),
        in_specs=[
            pl.BlockSpec((_ROWS, _BLOCK_W), lambda j: (0, j)),
            pl.BlockSpec((_ROWS, 1), lambda j: (0, 0)),
        ],
        out_specs=pl.BlockSpec((1, 1, 1), lambda j: (j, 0, 0)),
        out_shape=jax.ShapeDtypeStruct((_NUM_BLOCKS, 1, 1), jnp.float32),
        compiler_params=pltpu.CompilerParams(
            dimension_semantics=("parallel",),
        ),
    )(lsm, tgt)
    return -jnp.sum(partials) / _ROWS


# speedup vs baseline: 1.0054x; 1.0054x over previous
import jax
import jax.numpy as jnp
from jax.experimental import pallas as pl
from jax.experimental.pallas import tpu as pltpu

_N_CLASSES = 100000
_SMOOTHING = 0.1
_CONFIDENCE = 1.0 - _SMOOTHING
_EPS = _SMOOTHING / (_N_CLASSES - 1)

_ROWS = 1024
_BLOCK_R = 32
_NUM_BLOCKS = _ROWS // _BLOCK_R


def _loss_kernel(lsm_ref, tgt_ref, out_ref):
    blk = lsm_ref[...]
    col = jax.lax.broadcasted_iota(jnp.int32, (_BLOCK_R, _N_CLASSES), 1)
    s = jnp.sum(blk)
    tgt = tgt_ref[...]
    g = jnp.sum(jnp.where(col == tgt, blk, 0.0))
    out_ref[...] = jnp.reshape(_EPS * s + (_CONFIDENCE - _EPS) * g, (1, 1, 1))


def kernel(lsm, target):
    tgt = target.astype(jnp.int32).reshape(_ROWS, 1)
    partials = pl.pallas_call(
        _loss_kernel,
        grid=(_NUM_BLOCKS,),
        in_specs=[
            pl.BlockSpec((_BLOCK_R, _N_CLASSES), lambda j: (j, 0)),
            pl.BlockSpec((_BLOCK_R, 1), lambda j: (j, 0)),
        ],
        out_specs=pl.BlockSpec((1, 1, 1), lambda j: (j, 0, 0)),
        out_shape=jax.ShapeDtypeStruct((_NUM_BLOCKS, 1, 1), jnp.float32),
        compiler_params=pltpu.CompilerParams(
            dimension_semantics=("parallel",),
        ),
    )(lsm, tgt)
    return -jnp.sum(partials) / _ROWS
